# baseline (speedup 1.0000x reference)
import functools

import jax
import jax.numpy as jnp
from jax.experimental import pallas as pl
from jax.experimental.pallas import tpu as pltpu

B = 8
N = 10000
D = 128
NUM_BLOCKS = 10
BN = N // NUM_BLOCKS


def _fused_kernel(x_ref, w0_ref, b0_ref, w1_ref, b1_ref, wo_ref, bo_ref,
                  out_ref, acc_ref):
    step = pl.program_id(0)

    @pl.when(step == 0)
    def _init():
        acc_ref[...] = jnp.zeros_like(acc_ref)

    x = x_ref[...].reshape(B * BN, D)
    h = jnp.maximum(jnp.dot(x, w0_ref[...]) + b0_ref[...], 0.0)
    h = jnp.maximum(jnp.dot(h, w1_ref[...]) + b1_ref[...], 0.0)
    acc_ref[...] += h.reshape(B, BN, D).sum(axis=1)

    @pl.when(step == NUM_BLOCKS - 1)
    def _finish():
        pooled = acc_ref[...] * (1.0 / N)
        out_ref[...] = jnp.dot(pooled, wo_ref[...]) + bo_ref[...]


@functools.partial(jax.jit, static_argnames=("interpret",))
def _run(inputs, W0, b0, W1, b1, W_out, b_out, interpret=False):
    full = lambda shape: pl.BlockSpec(shape, lambda i: (0,) * len(shape))
    return pl.pallas_call(
        _fused_kernel,
        grid=(NUM_BLOCKS,),
        in_specs=[
            pl.BlockSpec((B, BN, D), lambda i: (0, i, 0)),
            full((D, D)),
            full((1, D)),
            full((D, D)),
            full((1, D)),
            full((D, D)),
            full((1, D)),
        ],
        out_specs=full((B, D)),
        out_shape=jax.ShapeDtypeStruct((B, D), jnp.float32),
        scratch_shapes=[pltpu.VMEM((B, D), jnp.float32)],
        interpret=interpret,
    )(inputs, W0, b0.reshape(1, D), W1, b1.reshape(1, D),
      W_out, b_out.reshape(1, D))


def kernel(inputs, W0, b0, W1, b1, W_out, b_out):
    return _run(inputs, W0, b0, W1, b1, W_out, b_out)
